# baseline (speedup 1.0000x reference)
import functools

import jax
import jax.numpy as jnp
from jax import lax
from jax.experimental import pallas as pl
from jax.experimental.pallas import tpu as pltpu
from jax.experimental.pallas import tpu_sc as plsc

BATCH = 16384
EMB = 64
VOCAB = 1000000
NC, NS = 2, 16
NW = NC * NS
BPW = BATCH // NW
CHUNK = 128
NCH = BPW // CHUNK

HALF = 500224
NLO = HALF // 128
NHI = (VOCAB - 64 - HALF) // 128
ROWCAP = VOCAB - 64
NTASK = (NLO + NW - 1) // NW
PR = 2 * EMB

_mesh = plsc.VectorSubcoreMesh(core_axis_name="c", subcore_axis_name="s")


@functools.partial(
    pl.kernel,
    mesh=_mesh,
    out_type=jax.ShapeDtypeStruct((HALF * PR,), jnp.float32),
    scratch_types=[
        pltpu.VMEM((EMB, CHUNK), jnp.float32),
        pltpu.VMEM((EMB, CHUNK), jnp.float32),
        pltpu.VMEM((CHUNK * PR,), jnp.float32),
        pltpu.SemaphoreType.DMA,
    ],
    compiler_params=pltpu.CompilerParams(needs_layout_passes=False),
)
def _pack(tt_hbm, pairs_hbm, lo_v, hi_v, ob_v, sem):
    wid = lax.axis_index("s") * NC + lax.axis_index("c")

    def task(t, carry):
        b = wid + t * NW

        @pl.when(b < NLO)
        def _():
            col_lo = pl.multiple_of(b * CHUNK, CHUNK)
            bh = jnp.minimum(b, NHI - 1)
            col_hi = pl.multiple_of(HALF + bh * CHUNK, CHUNK)
            c1 = pltpu.async_copy(tt_hbm.at[:, pl.ds(col_lo, CHUNK)], lo_v, sem)
            c2 = pltpu.async_copy(tt_hbm.at[:, pl.ds(col_hi, CHUNK)], hi_v, sem)
            c1.wait()
            c2.wait()

            iota = lax.broadcasted_iota(jnp.int32, (16,), 0)

            def col(c, carry2):
                for g in range(CHUNK // 16):
                    tgt = (iota + g * 16) * PR + c
                    plsc.store_scatter(ob_v, [tgt], lo_v[c, pl.ds(g * 16, 16)])
                    plsc.store_scatter(
                        ob_v, [tgt + EMB], hi_v[c, pl.ds(g * 16, 16)]
                    )
                return carry2

            lax.fori_loop(0, EMB, col, 0)
            prow = pl.multiple_of(b * (CHUNK * PR), CHUNK * PR)
            pltpu.sync_copy(ob_v, pairs_hbm.at[pl.ds(prow, CHUNK * PR)])

        return carry

    lax.fori_loop(0, NTASK, task, 0)


def _aux_body(tt_ref, out_ref):
    out_ref[:, 0:EMB] = jnp.transpose(tt_ref[...], (1, 0))[0:EMB, :]


_aux = pl.pallas_call(
    _aux_body,
    grid=(1,),
    in_specs=[pl.BlockSpec((EMB, CHUNK), lambda i: (0, ROWCAP // CHUNK))],
    out_specs=pl.BlockSpec((EMB, PR), lambda i: (0, 0)),
    out_shape=jax.ShapeDtypeStruct((EMB, PR), jnp.float32),
)


@functools.partial(
    pl.kernel,
    mesh=_mesh,
    out_type=jax.ShapeDtypeStruct((BATCH, PR), jnp.float32),
    scratch_types=[
        pltpu.VMEM((NCH, CHUNK), jnp.int32),
        pltpu.VMEM((CHUNK,), jnp.int32),
        pltpu.VMEM((CHUNK * PR,), jnp.float32),
        pltpu.VMEM((CHUNK, EMB), jnp.float32),
        pltpu.VMEM((CHUNK, PR), jnp.float32),
        pltpu.SemaphoreType.DMA,
        pltpu.SemaphoreType.DMA,
    ],
)
def _encode(idx_hbm, w2v_hbm, pairs_hbm, aux_hbm, out_hbm, idx_v, off_v, emb_v,
            w2v_v, buf_v, gsem, wsem):
    wid = lax.axis_index("s") * NC + lax.axis_index("c")
    base = wid * BPW
    pltpu.sync_copy(idx_hbm.at[pl.ds(wid * NCH, NCH)], idx_v)
    for j in range(NCH):
        cbase = base + j * CHUNK
        wcopy = pltpu.async_copy(w2v_hbm.at[pl.ds(cbase, CHUNK)], w2v_v, wsem)

        def issue(g, carry):
            vec = idx_v[j, pl.ds(g * 16, 16)]
            aux_f = 1 + ((vec - ROWCAP) >> 31)
            hi_f = (1 + ((vec - HALF) >> 31)) * (1 - aux_f)
            off_v[pl.ds(g * 16, 16)] = hi_f * EMB
            rows = vec - hi_f * HALF - aux_f * ROWCAP
            for k in range(16):
                rk = rows[k]
                ak = aux_f[k]
                dst = emb_v.at[pl.ds((g * 16 + k) * PR, PR)]

                @pl.when(ak == 0)
                def _():
                    pltpu.make_async_copy(
                        pairs_hbm.at[pl.ds(rk * PR, PR)], dst, gsem
                    ).start()

                @pl.when(ak == 1)
                def _():
                    pltpu.make_async_copy(
                        aux_hbm.at[pl.ds(rk * PR, PR)], dst, gsem
                    ).start()

            return carry

        lax.fori_loop(0, CHUNK // 16, issue, 0)

        def drain(r, carry):
            pltpu.make_async_copy(
                pairs_hbm.at[pl.ds(0, PR)], emb_v.at[pl.ds(0, PR)], gsem
            ).wait()
            return carry

        lax.fori_loop(0, CHUNK, drain, 0)
        wcopy.wait()

        def body(g, carry):
            offs = off_v[pl.ds(g * 16, 16)]
            for k in range(16):
                r = g * 16 + k
                o = offs[k]
                for c in range(EMB // 16):
                    buf_v[r, pl.ds(c * 16, 16)] = emb_v[pl.ds(r * PR + o + c * 16, 16)]
                    buf_v[r, pl.ds(EMB + c * 16, 16)] = w2v_v[r, pl.ds(c * 16, 16)]
            return carry

        lax.fori_loop(0, CHUNK // 16, body, 0)
        pltpu.sync_copy(buf_v, out_hbm.at[pl.ds(cbase, CHUNK)])


def kernel(category_id, word2vec, emb_table):
    idx = category_id.astype(jnp.int32).reshape(NW * NCH, CHUNK)
    tt = emb_table.T
    pairs = _pack(tt)
    aux = _aux(tt).reshape(EMB * PR)
    return _encode(idx, word2vec, pairs, aux)


# speedup vs baseline: 4.0440x; 4.0440x over previous
import functools

import jax
import jax.numpy as jnp
from jax import lax
from jax.experimental import pallas as pl
from jax.experimental.pallas import tpu as pltpu
from jax.experimental.pallas import tpu_sc as plsc

BATCH = 16384
EMB = 64
NC, NS = 2, 16
NW = NC * NS
BPW = BATCH // NW
CHUNK = 128
NCH = BPW // CHUNK

_mesh = plsc.VectorSubcoreMesh(core_axis_name="c", subcore_axis_name="s")


@functools.partial(
    pl.kernel,
    mesh=_mesh,
    out_type=jax.ShapeDtypeStruct((BATCH, 2 * EMB), jnp.float32),
    scratch_types=[
        pltpu.VMEM((NCH, CHUNK), jnp.int32),
        pltpu.VMEM((CHUNK, EMB), jnp.float32),
        pltpu.VMEM((CHUNK, EMB), jnp.float32),
        pltpu.VMEM((CHUNK, 2 * EMB), jnp.float32),
        pltpu.SemaphoreType.DMA,
        pltpu.SemaphoreType.DMA,
    ],
)
def _encode(idx_hbm, w2v_hbm, table_hbm, out_hbm, idx_v, emb_v, w2v_v, buf_v,
            gsem, wsem):
    wid = lax.axis_index("s") * NC + lax.axis_index("c")
    base = wid * BPW
    pltpu.sync_copy(idx_hbm.at[pl.ds(wid * NCH, NCH)], idx_v)
    for j in range(NCH):
        cbase = base + j * CHUNK
        wcopy = pltpu.async_copy(w2v_hbm.at[pl.ds(cbase, CHUNK)], w2v_v, wsem)

        def issue(g, carry):
            vec = idx_v[j, pl.ds(g * 16, 16)]
            for k in range(16):
                pltpu.make_async_copy(
                    table_hbm.at[pl.ds(vec[k], 1)],
                    emb_v.at[pl.ds(g * 16 + k, 1)],
                    gsem,
                ).start()
            return carry

        lax.fori_loop(0, CHUNK // 16, issue, 0)

        def drain(r, carry):
            pltpu.make_async_copy(
                table_hbm.at[pl.ds(0, 1)], emb_v.at[pl.ds(0, 1)], gsem
            ).wait()
            return carry

        lax.fori_loop(0, CHUNK, drain, 0)
        wcopy.wait()

        def body(r, carry):
            for c in range(EMB // 16):
                buf_v[r, pl.ds(c * 16, 16)] = emb_v[r, pl.ds(c * 16, 16)]
                buf_v[r, pl.ds(EMB + c * 16, 16)] = w2v_v[r, pl.ds(c * 16, 16)]
            return carry

        lax.fori_loop(0, CHUNK, body, 0)
        pltpu.sync_copy(buf_v, out_hbm.at[pl.ds(cbase, CHUNK)])


def kernel(category_id, word2vec, emb_table):
    idx = category_id.astype(jnp.int32).reshape(NW * NCH, CHUNK)
    return _encode(idx, word2vec, emb_table)
